# baseline (speedup 1.0000x reference)
import functools

import jax
import jax.numpy as jnp
from jax import lax
from jax.experimental import pallas as pl
from jax.experimental.pallas import tpu as pltpu
from jax.experimental.pallas import tpu_sc as plsc

B = 16384
K = 64
NC = 2
NS = 16
NW = NC * NS
BPW = B // NW
CHUNK = 128
NCHUNK = BPW // CHUNK
GROUPS = CHUNK // 16

_mesh = plsc.VectorSubcoreMesh(core_axis_name="c", subcore_axis_name="s")

_GATHER_DNUMS = lax.GatherDimensionNumbers(
    offset_dims=(), collapsed_slice_dims=(0,), start_index_map=(0,))


def _permute(x, idx):
    return lax.gather(x, idx[:, None], _GATHER_DNUMS, (1,),
                      mode=lax.GatherScatterMode.PROMISE_IN_BOUNDS)


@functools.partial(
    pl.kernel,
    out_type=jax.ShapeDtypeStruct((NW, NCHUNK, CHUNK), jnp.float32),
    mesh=_mesh,
    compiler_params=pltpu.CompilerParams(use_tc_tiling_on_sc=False),
    scratch_types=[
        pltpu.VMEM((NCHUNK, CHUNK), jnp.int32),
        pltpu.VMEM((NCHUNK, CHUNK), jnp.int32),
        pltpu.VMEM((BPW, K), jnp.float32),
        pltpu.VMEM((BPW, K), jnp.float32),
        pltpu.VMEM((NCHUNK, CHUNK), jnp.float32),
        pltpu.VMEM((NCHUNK, CHUNK), jnp.float32),
        pltpu.VMEM((NCHUNK, CHUNK), jnp.float32),
        pltpu.SemaphoreType.DMA,
    ],
)
def _mf_sc(user_hbm, item_hbm, ue_hbm, ie_hbm, ub_hbm, ib_hbm, out_hbm,
           idx_u, idx_i, u_rows, i_rows, bu_v, bi_v, out_v, sem):
    wid = lax.axis_index("s") * NC + lax.axis_index("c")

    pltpu.sync_copy(user_hbm.at[wid], idx_u)
    pltpu.sync_copy(item_hbm.at[wid], idx_i)

    copies = []
    for c in range(NCHUNK):
        copies.append(pltpu.async_copy(
            ue_hbm.at[idx_u.at[c]], u_rows.at[pl.ds(c * CHUNK, CHUNK)], sem))
        copies.append(pltpu.async_copy(
            ie_hbm.at[idx_i.at[c]], i_rows.at[pl.ds(c * CHUNK, CHUNK)], sem))
        copies.append(pltpu.async_copy(ub_hbm.at[idx_u.at[c]], bu_v.at[c], sem))
        copies.append(pltpu.async_copy(ib_hbm.at[idx_i.at[c]], bi_v.at[c], sem))
    for cp in copies:
        cp.wait()

    lane = lax.iota(jnp.int32, 16)

    for c in range(NCHUNK):

        def group_body(g, _, c=c):
            base_e = c * CHUNK + g * 16
            res = jnp.zeros((16,), jnp.float32)
            for j in range(16):
                e = base_e + j
                acc = jnp.zeros((16,), jnp.float32)
                for t in range(K // 16):
                    u_chunk = u_rows[e, pl.ds(t * 16, 16)]
                    i_chunk = i_rows[e, pl.ds(t * 16, 16)]
                    acc = acc + u_chunk * i_chunk
                for sh in (1, 2, 4, 8):
                    acc = acc + _permute(acc, lane ^ sh)
                res = jnp.where(lane == j, acc, res)
            res = res + bu_v[c, pl.ds(g * 16, 16)] + bi_v[c, pl.ds(g * 16, 16)]
            out_v[c, pl.ds(g * 16, 16)] = res
            return _

        lax.fori_loop(0, GROUPS, group_body, 0)

    pltpu.sync_copy(out_v, out_hbm.at[wid])


def kernel(user, item, user_embedding, item_embedding, user_bias, item_bias):
    user = user.astype(jnp.int32).reshape(NW, NCHUNK, CHUNK)
    item = item.astype(jnp.int32).reshape(NW, NCHUNK, CHUNK)
    ub = user_bias.reshape(-1)
    ib = item_bias.reshape(-1)
    out = _mf_sc(user, item, user_embedding, item_embedding, ub, ib)
    return out.reshape(B)
